# baseline (speedup 1.0000x reference)
import functools

import jax
import jax.numpy as jnp
from jax import lax
from jax.experimental import pallas as pl
from jax.experimental.pallas import tpu as pltpu
from jax.experimental.pallas import tpu_sc as plsc

NC = 2
NS = 16
LANES = 16

V = 10000
D = 128
E = 320000
NW = NC * NS
EPW = E // NW
CHUNK = 40
CHUNKS = EPW // CHUNK
NBUF = 4
NIDX = 8
VP = 10240
ROWS_PER_SUB = VP // NS


def _rspmm_body(xs, zs, eiv, zinit, out, acc, *scr):
    xr = scr[0:NBUF]
    zr = scr[NBUF:2 * NBUF]
    ibuf = scr[2 * NBUF:2 * NBUF + NIDX]
    dstb = scr[2 * NBUF + NIDX:2 * NBUF + NIDX + NBUF]
    relb = scr[2 * NBUF + NIDX + NBUF:2 * NBUF + NIDX + 2 * NBUF]
    srcb = scr[2 * NBUF + NIDX + 2 * NBUF:2 * NBUF + NIDX + 3 * NBUF]
    base_s = 5 * NBUF + NIDX
    semg = scr[base_s:base_s + NBUF]
    sems = scr[base_s + NBUF:base_s + 2 * NBUF]
    semi = scr[base_s + 2 * NBUF:base_s + 2 * NBUF + NIDX]

    c = lax.axis_index("c")
    s = lax.axis_index("s")
    wid = c * NS + s
    base3 = wid * EPW * 3

    pltpu.sync_copy(zinit, acc.at[pl.ds(s * ROWS_PER_SUB, ROWS_PER_SUB)])
    plsc.subcore_barrier()

    lanes = lax.iota(jnp.int32, LANES)
    i3 = lanes * 3
    head8 = lanes < 8

    def start_idx(j, q):
        off = base3 + j * (3 * CHUNK)
        pltpu.async_copy(eiv.at[pl.ds(off, 3 * CHUNK)], ibuf[q], semi[q])

    def launch_rows(q, r):
        pltpu.make_async_copy(eiv.at[pl.ds(0, 3 * CHUNK)], ibuf[q],
                              semi[q]).wait()
        for col, buf in ((0, dstb[r]), (1, relb[r]), (2, srcb[r])):
            g0 = plsc.load_gather(ibuf[q], [i3 + col])
            g1 = plsc.load_gather(ibuf[q], [i3 + (48 + col)])
            g2 = plsc.load_gather(ibuf[q], [i3 + (96 + col)], mask=head8)
            buf[pl.ds(0, LANES)] = g0
            buf[pl.ds(LANES, LANES)] = g1
            plsc.store_scatter(buf, [lanes + 2 * LANES], g2, mask=head8)
        pltpu.async_copy(xs.at[srcb[r]], xr[r], semg[r])
        pltpu.async_copy(zs.at[relb[r]], zr[r], semg[r])

    def wait_gather(r):
        pltpu.make_async_copy(xs.at[srcb[r]], xr[r], semg[r]).wait()
        pltpu.make_async_copy(zs.at[relb[r]], zr[r], semg[r]).wait()

    for j in range(NIDX - 1):
        start_idx(j, j)
    for j in range(NBUF - 1):
        launch_rows(j, j)

    def emit_iter(j, u):
        r = u % NBUF
        q = u % NIDX

        @pl.when(j < CHUNKS)
        def _steady():
            wait_gather(r)

            @plsc.parallel_loop(0, CHUNK, unroll=2)
            def mul_row(rr):
                sls = [pl.ds(jj * LANES, LANES) for jj in range(D // LANES)]
                xvs = [xr[r][rr, sl] for sl in sls]
                zvs = [zr[r][rr, sl] for sl in sls]
                prods = [a * b for a, b in zip(xvs, zvs)]
                for sl, p in zip(sls, prods):
                    xr[r][rr, sl] = p
            pltpu.async_copy(xr[r], acc.at[dstb[r]], sems[r], add=True)

        @pl.when(jnp.logical_and(j >= 1, j <= CHUNKS))
        def _drain():
            rp = (u + NBUF - 1) % NBUF
            pltpu.make_async_copy(xr[rp], acc.at[dstb[rp]], sems[rp]).wait()

        @pl.when(j + NIDX - 1 < CHUNKS)
        def _pref_idx():
            start_idx(j + NIDX - 1, (u + NIDX - 1) % NIDX)

        @pl.when(j + NBUF - 1 < CHUNKS)
        def _pref_rows():
            launch_rows((u + NBUF - 1) % NIDX, (u + NBUF - 1) % NBUF)

    niter = (CHUNKS + 2 * NIDX - 1) // NIDX

    def outer(k, carry):
        for u in range(NIDX):
            emit_iter(k * NIDX + u, u)
        return carry

    lax.fori_loop(0, niter, outer, 0, unroll=False)
    plsc.subcore_barrier()

    pltpu.sync_copy(acc.at[pl.ds(s * ROWS_PER_SUB, ROWS_PER_SUB)],
                    out.at[c, pl.ds(s * ROWS_PER_SUB, ROWS_PER_SUB)])


@functools.cache
def _get_rspmm_sc():
    scratch = (
        [pltpu.VMEM((CHUNK, D), jnp.float32) for _ in range(2 * NBUF)]
        + [pltpu.VMEM((3 * CHUNK,), jnp.int32) for _ in range(NIDX)]
        + [pltpu.VMEM((CHUNK,), jnp.int32) for _ in range(3 * NBUF)]
        + [pltpu.SemaphoreType.DMA for _ in range(2 * NBUF + NIDX)]
    )
    return functools.partial(
        pl.kernel,
        compiler_params=pltpu.CompilerParams(needs_layout_passes=False),
        out_type=jax.ShapeDtypeStruct((NC, VP, D), jnp.float32),
        mesh=plsc.VectorSubcoreMesh(core_axis_name="c", subcore_axis_name="s",
                                    num_cores=NC, num_subcores=NS),
        scratch_types=[pltpu.VMEM_SHARED((VP, D), jnp.float32)] + scratch,
    )(_rspmm_body)


BLK = 2000


def _dense_body(p0, p1, xb, W1, b1, W2, b2, alpha, lns, lnb, out):
    xv = xb[0]
    h = p0[0] + p1[0] + alpha[...] * xv
    h = lax.dot_general(h, W1[...], (((1,), (1,)), ((), ())),
                        preferred_element_type=jnp.float32) + b1[...]
    h = jnp.maximum(h, 0.0)
    h = lax.dot_general(h, W2[...], (((1,), (1,)), ((), ())),
                        preferred_element_type=jnp.float32) + b2[...]
    mu = jnp.mean(h, axis=-1, keepdims=True)
    hc = h - mu
    var = jnp.mean(hc * hc, axis=-1, keepdims=True)
    hn = hc * lax.rsqrt(var + 1e-5) * lns[...] + lnb[...]
    out[0] = hn + xv


def _dense_tc(partials, x, W1, b1, W2, b2, alpha, lns, lnb):
    full = lambda shape: pl.BlockSpec(shape, lambda i: (0,) * len(shape))
    grid = V // BLK
    return pl.pallas_call(
        _dense_body,
        grid=(grid,),
        in_specs=[
            pl.BlockSpec((1, BLK, D), lambda i: (0, i, 0)),
            pl.BlockSpec((1, BLK, D), lambda i: (1, i, 0)),
            pl.BlockSpec((1, BLK, D), lambda i: (0, i, 0)),
            full((D, D)), full((1, D)), full((D, D)), full((1, D)),
            full((1, D)), full((1, D)), full((1, D)),
        ],
        out_specs=pl.BlockSpec((1, BLK, D), lambda i: (0, i, 0)),
        out_shape=jax.ShapeDtypeStruct((1, V, D), jnp.float32),
    )(partials, partials, x, W1, b1, W2, b2, alpha, lns, lnb)


def kernel(x, z, edge_index, W1, b1, W2, b2, alpha, ln_scale, ln_bias):
    xs = x[0]
    zs = z[0]
    eiv = edge_index.astype(jnp.int32).reshape(-1)
    zinit = jnp.zeros((ROWS_PER_SUB, D), jnp.float32)
    partials = _get_rspmm_sc()(xs, zs, eiv, zinit)
    return _dense_tc(partials, x, W1, b1.reshape(1, D), W2, b2.reshape(1, D),
                     alpha, ln_scale.reshape(1, D), ln_bias.reshape(1, D))


# speedup vs baseline: 1.8012x; 1.8012x over previous
import functools

import jax
import jax.numpy as jnp
from jax import lax
from jax.experimental import pallas as pl
from jax.experimental.pallas import tpu as pltpu
from jax.experimental.pallas import tpu_sc as plsc

NC = 2
NS = 16
LANES = 16

V = 10000
D = 128
E = 320000
NW = NC * NS
EPW = E // NW
CHUNK = 40
CHUNKS = EPW // CHUNK
NBUF = 4
NIDX = 8
VP = 10240
ROWS_PER_SUB = VP // NS


def _rspmm_body(xs, zs, dsts, rels, srcs, zinit, out, acc, *scr):
    xr = scr[0:NBUF]
    zr = scr[NBUF:2 * NBUF]
    dstb = scr[2 * NBUF:2 * NBUF + NIDX]
    relb = scr[2 * NBUF + NIDX:2 * NBUF + 2 * NIDX]
    srcb = scr[2 * NBUF + 2 * NIDX:2 * NBUF + 3 * NIDX]
    base_s = 2 * NBUF + 3 * NIDX
    semg = scr[base_s:base_s + NBUF]
    sems = scr[base_s + NBUF:base_s + 2 * NBUF]
    semi = scr[base_s + 2 * NBUF:base_s + 2 * NBUF + NIDX]

    c = lax.axis_index("c")
    s = lax.axis_index("s")
    wid = c * NS + s
    base = wid * EPW

    pltpu.sync_copy(zinit, acc.at[pl.ds(s * ROWS_PER_SUB, ROWS_PER_SUB)])
    plsc.subcore_barrier()

    def start_idx(j, q):
        off = base + j * CHUNK
        pltpu.async_copy(dsts.at[pl.ds(off, CHUNK)], dstb[q], semi[q])
        pltpu.async_copy(rels.at[pl.ds(off, CHUNK)], relb[q], semi[q])
        pltpu.async_copy(srcs.at[pl.ds(off, CHUNK)], srcb[q], semi[q])

    def wait_idx(q):
        pltpu.make_async_copy(dsts.at[pl.ds(0, CHUNK)], dstb[q], semi[q]).wait()
        pltpu.make_async_copy(rels.at[pl.ds(0, CHUNK)], relb[q], semi[q]).wait()
        pltpu.make_async_copy(srcs.at[pl.ds(0, CHUNK)], srcb[q], semi[q]).wait()

    def start_gather(q, r):
        pltpu.async_copy(xs.at[srcb[q]], xr[r], semg[r])
        pltpu.async_copy(zs.at[relb[q]], zr[r], semg[r])

    def wait_gather(q, r):
        pltpu.make_async_copy(xs.at[srcb[q]], xr[r], semg[r]).wait()
        pltpu.make_async_copy(zs.at[relb[q]], zr[r], semg[r]).wait()

    for j in range(NIDX - 1):
        start_idx(j, j)
    for j in range(NBUF - 1):
        wait_idx(j)
        start_gather(j, j)

    def emit_iter(j, u):
        r = u % NBUF
        q = u % NIDX

        @pl.when(j < CHUNKS)
        def _steady():
            wait_gather(q, r)

            @plsc.parallel_loop(0, CHUNK, unroll=2)
            def mul_row(rr):
                sls = [pl.ds(jj * LANES, LANES) for jj in range(D // LANES)]
                xvs = [xr[r][rr, sl] for sl in sls]
                zvs = [zr[r][rr, sl] for sl in sls]
                prods = [a * b for a, b in zip(xvs, zvs)]
                for sl, p in zip(sls, prods):
                    xr[r][rr, sl] = p
            pltpu.async_copy(xr[r], acc.at[dstb[q]], sems[r], add=True)

        @pl.when(jnp.logical_and(j >= 1, j <= CHUNKS))
        def _drain():
            rp = (u + NBUF - 1) % NBUF
            qp = (u + NIDX - 1) % NIDX
            pltpu.make_async_copy(xr[rp], acc.at[dstb[qp]], sems[rp]).wait()

        @pl.when(j + NIDX - 1 < CHUNKS)
        def _pref_idx():
            start_idx(j + NIDX - 1, (u + NIDX - 1) % NIDX)

        @pl.when(j + NBUF - 1 < CHUNKS)
        def _pref_rows():
            qn = (u + NBUF - 1) % NIDX
            rn = (u + NBUF - 1) % NBUF
            wait_idx(qn)
            start_gather(qn, rn)

    niter = (CHUNKS + 2 * NIDX - 1) // NIDX

    def outer(k, carry):
        for u in range(NIDX):
            emit_iter(k * NIDX + u, u)
        return carry

    lax.fori_loop(0, niter, outer, 0, unroll=False)
    plsc.subcore_barrier()

    pltpu.sync_copy(acc.at[pl.ds(s * ROWS_PER_SUB, ROWS_PER_SUB)],
                    out.at[c, pl.ds(s * ROWS_PER_SUB, ROWS_PER_SUB)])


@functools.cache
def _get_rspmm_sc():
    scratch = (
        [pltpu.VMEM((CHUNK, D), jnp.float32) for _ in range(2 * NBUF)]
        + [pltpu.VMEM((CHUNK,), jnp.int32) for _ in range(3 * NIDX)]
        + [pltpu.SemaphoreType.DMA for _ in range(2 * NBUF + NIDX)]
    )
    return functools.partial(
        pl.kernel,
        compiler_params=pltpu.CompilerParams(needs_layout_passes=False),
        out_type=jax.ShapeDtypeStruct((NC, VP, D), jnp.float32),
        mesh=plsc.VectorSubcoreMesh(core_axis_name="c", subcore_axis_name="s",
                                    num_cores=NC, num_subcores=NS),
        scratch_types=[pltpu.VMEM_SHARED((VP, D), jnp.float32)] + scratch,
    )(_rspmm_body)


BLK = 2000


def _dense_body(p0, p1, xb, W1, b1, W2, b2, alpha, lns, lnb, out):
    xv = xb[0]
    h = p0[0] + p1[0] + alpha[...] * xv
    h = lax.dot_general(h, W1[...], (((1,), (1,)), ((), ())),
                        preferred_element_type=jnp.float32) + b1[...]
    h = jnp.maximum(h, 0.0)
    h = lax.dot_general(h, W2[...], (((1,), (1,)), ((), ())),
                        preferred_element_type=jnp.float32) + b2[...]
    mu = jnp.mean(h, axis=-1, keepdims=True)
    hc = h - mu
    var = jnp.mean(hc * hc, axis=-1, keepdims=True)
    hn = hc * lax.rsqrt(var + 1e-5) * lns[...] + lnb[...]
    out[0] = hn + xv


def _dense_tc(partials, x, W1, b1, W2, b2, alpha, lns, lnb):
    full = lambda shape: pl.BlockSpec(shape, lambda i: (0,) * len(shape))
    grid = V // BLK
    return pl.pallas_call(
        _dense_body,
        grid=(grid,),
        in_specs=[
            pl.BlockSpec((1, BLK, D), lambda i: (0, i, 0)),
            pl.BlockSpec((1, BLK, D), lambda i: (1, i, 0)),
            pl.BlockSpec((1, BLK, D), lambda i: (0, i, 0)),
            full((D, D)), full((1, D)), full((D, D)), full((1, D)),
            full((1, D)), full((1, D)), full((1, D)),
        ],
        out_specs=pl.BlockSpec((1, BLK, D), lambda i: (0, i, 0)),
        out_shape=jax.ShapeDtypeStruct((1, V, D), jnp.float32),
    )(partials, partials, x, W1, b1, W2, b2, alpha, lns, lnb)


def kernel(x, z, edge_index, W1, b1, W2, b2, alpha, ln_scale, ln_bias):
    xs = x[0]
    zs = z[0]
    ei = edge_index.astype(jnp.int32)
    dsts = ei[:, 0]
    rels = ei[:, 1]
    srcs = ei[:, 2]
    zinit = jnp.zeros((ROWS_PER_SUB, D), jnp.float32)
    partials = _get_rspmm_sc()(xs, zs, dsts, rels, srcs, zinit)
    return _dense_tc(partials, x, W1, b1.reshape(1, D), W2, b2.reshape(1, D),
                     alpha, ln_scale.reshape(1, D), ln_bias.reshape(1, D))


# speedup vs baseline: 1.9205x; 1.0662x over previous
import functools

import jax
import jax.numpy as jnp
from jax import lax
from jax.experimental import pallas as pl
from jax.experimental.pallas import tpu as pltpu
from jax.experimental.pallas import tpu_sc as plsc

NC = 2
NS = 16
LANES = 16

V = 10000
D = 128
E = 320000
NW = NC * NS
EPW = E // NW
CHUNK = 40
CHUNKS = EPW // CHUNK
NBUF = 4
NIDX = 8
VP = 10240
ROWS_PER_SUB = VP // NS


def _rspmm_body(xs, zs, eitf, zinit, out, acc, *scr):
    xr = scr[0:NBUF]
    zr = scr[NBUF:2 * NBUF]
    dstb = scr[2 * NBUF:2 * NBUF + NIDX]
    relb = scr[2 * NBUF + NIDX:2 * NBUF + 2 * NIDX]
    srcb = scr[2 * NBUF + 2 * NIDX:2 * NBUF + 3 * NIDX]
    base_s = 2 * NBUF + 3 * NIDX
    semg = scr[base_s:base_s + NBUF]
    sems = scr[base_s + NBUF:base_s + 2 * NBUF]
    semi = scr[base_s + 2 * NBUF:base_s + 2 * NBUF + NIDX]

    c = lax.axis_index("c")
    s = lax.axis_index("s")
    wid = c * NS + s
    base = wid * EPW

    pltpu.sync_copy(zinit, acc.at[pl.ds(s * ROWS_PER_SUB, ROWS_PER_SUB)])
    plsc.subcore_barrier()

    def start_idx(j, q):
        off = base + j * CHUNK
        pltpu.async_copy(eitf.at[pl.ds(off, CHUNK)], dstb[q], semi[q])
        pltpu.async_copy(eitf.at[pl.ds(E + off, CHUNK)], relb[q], semi[q])
        pltpu.async_copy(eitf.at[pl.ds(2 * E + off, CHUNK)], srcb[q], semi[q])

    def wait_idx(q):
        pltpu.make_async_copy(eitf.at[pl.ds(0, CHUNK)], dstb[q], semi[q]).wait()
        pltpu.make_async_copy(eitf.at[pl.ds(0, CHUNK)], relb[q], semi[q]).wait()
        pltpu.make_async_copy(eitf.at[pl.ds(0, CHUNK)], srcb[q], semi[q]).wait()

    def start_gather(q, r):
        pltpu.async_copy(xs.at[srcb[q]], xr[r], semg[r])
        pltpu.async_copy(zs.at[relb[q]], zr[r], semg[r])

    def wait_gather(q, r):
        pltpu.make_async_copy(xs.at[srcb[q]], xr[r], semg[r]).wait()
        pltpu.make_async_copy(zs.at[relb[q]], zr[r], semg[r]).wait()

    for j in range(NIDX - 1):
        start_idx(j, j)
    for j in range(NBUF - 1):
        wait_idx(j)
        start_gather(j, j)

    def emit_iter(j, u):
        r = u % NBUF
        q = u % NIDX

        @pl.when(j < CHUNKS)
        def _steady():
            wait_gather(q, r)

            @plsc.parallel_loop(0, CHUNK, unroll=2)
            def mul_row(rr):
                sls = [pl.ds(jj * LANES, LANES) for jj in range(D // LANES)]
                xvs = [xr[r][rr, sl] for sl in sls]
                zvs = [zr[r][rr, sl] for sl in sls]
                prods = [a * b for a, b in zip(xvs, zvs)]
                for sl, p in zip(sls, prods):
                    xr[r][rr, sl] = p
            pltpu.async_copy(xr[r], acc.at[dstb[q]], sems[r], add=True)

        @pl.when(jnp.logical_and(j >= 1, j <= CHUNKS))
        def _drain():
            rp = (u + NBUF - 1) % NBUF
            qp = (u + NIDX - 1) % NIDX
            pltpu.make_async_copy(xr[rp], acc.at[dstb[qp]], sems[rp]).wait()

        @pl.when(j + NIDX - 1 < CHUNKS)
        def _pref_idx():
            start_idx(j + NIDX - 1, (u + NIDX - 1) % NIDX)

        @pl.when(j + NBUF - 1 < CHUNKS)
        def _pref_rows():
            qn = (u + NBUF - 1) % NIDX
            rn = (u + NBUF - 1) % NBUF
            wait_idx(qn)
            start_gather(qn, rn)

    niter = (CHUNKS + 2 * NIDX - 1) // NIDX

    def outer(k, carry):
        for u in range(NIDX):
            emit_iter(k * NIDX + u, u)
        return carry

    lax.fori_loop(0, niter, outer, 0, unroll=False)
    plsc.subcore_barrier()

    pltpu.sync_copy(acc.at[pl.ds(s * ROWS_PER_SUB, ROWS_PER_SUB)],
                    out.at[c, pl.ds(s * ROWS_PER_SUB, ROWS_PER_SUB)])


@functools.cache
def _get_rspmm_sc():
    scratch = (
        [pltpu.VMEM((CHUNK, D), jnp.float32) for _ in range(2 * NBUF)]
        + [pltpu.VMEM((CHUNK,), jnp.int32) for _ in range(3 * NIDX)]
        + [pltpu.SemaphoreType.DMA for _ in range(2 * NBUF + NIDX)]
    )
    return functools.partial(
        pl.kernel,
        out_type=jax.ShapeDtypeStruct((NC, VP, D), jnp.float32),
        mesh=plsc.VectorSubcoreMesh(core_axis_name="c", subcore_axis_name="s",
                                    num_cores=NC, num_subcores=NS),
        scratch_types=[pltpu.VMEM_SHARED((VP, D), jnp.float32)] + scratch,
    )(_rspmm_body)


BLK = 2000


def _dense_body(p0, p1, xb, W1, b1, W2, b2, alpha, lns, lnb, out):
    xv = xb[0]
    h = p0[0] + p1[0] + alpha[...] * xv
    h = lax.dot_general(h, W1[...], (((1,), (1,)), ((), ())),
                        preferred_element_type=jnp.float32) + b1[...]
    h = jnp.maximum(h, 0.0)
    h = lax.dot_general(h, W2[...], (((1,), (1,)), ((), ())),
                        preferred_element_type=jnp.float32) + b2[...]
    mu = jnp.mean(h, axis=-1, keepdims=True)
    hc = h - mu
    var = jnp.mean(hc * hc, axis=-1, keepdims=True)
    hn = hc * lax.rsqrt(var + 1e-5) * lns[...] + lnb[...]
    out[0] = hn + xv


def _dense_tc(partials, x, W1, b1, W2, b2, alpha, lns, lnb):
    full = lambda shape: pl.BlockSpec(shape, lambda i: (0,) * len(shape))
    grid = V // BLK
    return pl.pallas_call(
        _dense_body,
        grid=(grid,),
        in_specs=[
            pl.BlockSpec((1, BLK, D), lambda i: (0, i, 0)),
            pl.BlockSpec((1, BLK, D), lambda i: (1, i, 0)),
            pl.BlockSpec((1, BLK, D), lambda i: (0, i, 0)),
            full((D, D)), full((1, D)), full((D, D)), full((1, D)),
            full((1, D)), full((1, D)), full((1, D)),
        ],
        out_specs=pl.BlockSpec((1, BLK, D), lambda i: (0, i, 0)),
        out_shape=jax.ShapeDtypeStruct((1, V, D), jnp.float32),
    )(partials, partials, x, W1, b1, W2, b2, alpha, lns, lnb)


def kernel(x, z, edge_index, W1, b1, W2, b2, alpha, ln_scale, ln_bias):
    xs = x[0]
    zs = z[0]
    eitf = edge_index.astype(jnp.int32).T.reshape(-1)
    zinit = jnp.zeros((ROWS_PER_SUB, D), jnp.float32)
    partials = _get_rspmm_sc()(xs, zs, eitf, zinit)
    return _dense_tc(partials, x, W1, b1.reshape(1, D), W2, b2.reshape(1, D),
                     alpha, ln_scale.reshape(1, D), ln_bias.reshape(1, D))
